# baseline (speedup 1.0000x reference)
import jax
import jax.numpy as jnp
from jax import lax
from jax.experimental import pallas as pl
from jax.experimental.pallas import tpu as pltpu
from jax.experimental.pallas import tpu_sc as plsc

_GDN = lax.GatherDimensionNumbers(
    offset_dims=(), collapsed_slice_dims=(0,), start_index_map=(0,))

D = 128
L = 16
NC = 2
NS = 16
NW = NC * NS
B = 80


def _ip_body(xu, xm, eidx, out, idxu_all, idxm_all,
             u0, m0, u1, m1, obuf,
             su0, sm0, su1, sm1):
    wid = lax.axis_index("s") * NC + lax.axis_index("c")
    n_edges = out.shape[0]
    epw = n_edges // NW
    chunks = epw // B
    base = wid * epw

    pltpu.sync_copy(eidx.at[pl.ds(base, epw)], idxu_all)
    pltpu.sync_copy(eidx.at[pl.ds(n_edges + base, epw)], idxm_all)

    def issue(c, ub, mb, su, sm):
        o = c * B
        pltpu.async_copy(xu.at[idxu_all.at[pl.ds(o, B)]], ub, su)
        pltpu.async_copy(xm.at[idxm_all.at[pl.ds(o, B)]], mb, sm)

    def wait(ub, mb, su, sm):
        pltpu.make_async_copy(xu.at[idxu_all.at[pl.ds(0, B)]], ub, su).wait()
        pltpu.make_async_copy(xm.at[idxm_all.at[pl.ds(0, B)]], mb, sm).wait()

    lane = lax.iota(jnp.int32, L)
    last_lane = jnp.full((L, 1), L - 1, dtype=jnp.int32)

    def compute(c, ub, mb):
        def group_body(g, c2):
            e0 = g * L
            merged = jnp.zeros((L,), jnp.float32)
            for t in range(L):
                e = e0 + t
                a0 = ub[e, pl.ds(0, L)] * mb[e, pl.ds(0, L)]
                a1 = ub[e, pl.ds(L, L)] * mb[e, pl.ds(L, L)]
                for k in range(2, D // L, 2):
                    a0 = a0 + ub[e, pl.ds(k * L, L)] * mb[e, pl.ds(k * L, L)]
                    a1 = a1 + ub[e, pl.ds((k + 1) * L, L)] * mb[e, pl.ds((k + 1) * L, L)]
                cum = plsc.cumsum(a0 + a1)
                tot = lax.gather(cum, last_lane, _GDN, slice_sizes=(1,),
                                 mode=lax.GatherScatterMode.PROMISE_IN_BOUNDS)
                merged = jnp.where(lane == t, tot, merged)
            obuf[pl.ds(e0, L)] = merged
            return c2

        lax.fori_loop(0, B // L, group_body, 0)
        pltpu.sync_copy(obuf, out.at[pl.ds(base + c * B, B)])

    issue(0, u0, m0, su0, sm0)

    def pair_body(j, carry):
        c = 2 * j
        issue(c + 1, u1, m1, su1, sm1)
        wait(u0, m0, su0, sm0)
        compute(c, u0, m0)
        issue(c + 2, u0, m0, su0, sm0)
        wait(u1, m1, su1, sm1)
        compute(c + 1, u1, m1)
        return carry

    lax.fori_loop(0, (chunks - 1) // 2, pair_body, 0)
    wait(u0, m0, su0, sm0)
    compute(chunks - 1, u0, m0)


def kernel(x_user, x_movie, edge_label_index):
    n_edges = edge_label_index.shape[1]
    epw = n_edges // NW
    mesh = plsc.VectorSubcoreMesh(core_axis_name="c", subcore_axis_name="s")
    f = pl.kernel(
        _ip_body,
        out_type=jax.ShapeDtypeStruct((n_edges,), jnp.float32),
        mesh=mesh,
        compiler_params=pltpu.CompilerParams(needs_layout_passes=False),
        scratch_types=[
            pltpu.VMEM((epw,), jnp.int32),
            pltpu.VMEM((epw,), jnp.int32),
            pltpu.VMEM((B, D), jnp.float32),
            pltpu.VMEM((B, D), jnp.float32),
            pltpu.VMEM((B, D), jnp.float32),
            pltpu.VMEM((B, D), jnp.float32),
            pltpu.VMEM((B,), jnp.float32),
            pltpu.SemaphoreType.DMA,
            pltpu.SemaphoreType.DMA,
            pltpu.SemaphoreType.DMA,
            pltpu.SemaphoreType.DMA,
        ],
    )
    return f(x_user, x_movie, edge_label_index.reshape(2 * n_edges))


# speedup vs baseline: 1.0978x; 1.0978x over previous
import jax
import jax.numpy as jnp
from jax import lax
from jax.experimental import pallas as pl
from jax.experimental.pallas import tpu as pltpu
from jax.experimental.pallas import tpu_sc as plsc

_GDN = lax.GatherDimensionNumbers(
    offset_dims=(), collapsed_slice_dims=(0,), start_index_map=(0,))

D = 128
L = 16
NC = 2
NS = 16
NW = NC * NS
B = 80


def _ip_body(xu, xm, eidx, out, idxu_all, idxm_all,
             u0, m0, u1, m1, obuf,
             su0, sm0, su1, sm1):
    wid = lax.axis_index("s") * NC + lax.axis_index("c")
    n_edges = out.shape[0]
    epw = n_edges // NW
    chunks = epw // B
    base = wid * epw

    pltpu.sync_copy(eidx.at[pl.ds(base, epw)], idxu_all)
    pltpu.sync_copy(eidx.at[pl.ds(n_edges + base, epw)], idxm_all)

    def issue(c, ub, mb, su, sm):
        o = c * B
        pltpu.async_copy(xu.at[idxu_all.at[pl.ds(o, B)]], ub, su)
        pltpu.async_copy(xm.at[idxm_all.at[pl.ds(o, B)]], mb, sm)

    def wait(ub, mb, su, sm):
        pltpu.make_async_copy(xu.at[idxu_all.at[pl.ds(0, B)]], ub, su).wait()
        pltpu.make_async_copy(xm.at[idxm_all.at[pl.ds(0, B)]], mb, sm).wait()

    lane = lax.iota(jnp.int32, L)
    xor_idx = {x: (lane ^ x).reshape(L, 1) for x in (8, 4, 2, 1)}
    fold_mask = {x: (lane & x) == 0 for x in (8, 4, 2, 1)}

    def shuffle(v, x):
        return lax.gather(v, xor_idx[x], _GDN, slice_sizes=(1,),
                          mode=lax.GatherScatterMode.PROMISE_IN_BOUNDS)

    def combine(a, b, x):
        return jnp.where(fold_mask[x], a + shuffle(a, x), b + shuffle(b, x))

    def compute(c, ub, mb):
        def group_body(g, c2):
            e0 = g * L

            def leaf(t):
                e = e0 + t
                a0 = ub[e, pl.ds(0, L)] * mb[e, pl.ds(0, L)]
                a1 = ub[e, pl.ds(L, L)] * mb[e, pl.ds(L, L)]
                for k in range(2, D // L, 2):
                    a0 = a0 + ub[e, pl.ds(k * L, L)] * mb[e, pl.ds(k * L, L)]
                    a1 = a1 + ub[e, pl.ds((k + 1) * L, L)] * mb[e, pl.ds((k + 1) * L, L)]
                return a0 + a1

            def build(t0, size):
                if size == 1:
                    return leaf(t0)
                h = size // 2
                a = build(t0, h)
                b = build(t0 + h, h)
                return combine(a, b, {16: 8, 8: 4, 4: 2, 2: 1}[size])

            obuf[pl.ds(e0, L)] = build(0, L)
            return c2

        lax.fori_loop(0, B // L, group_body, 0)
        pltpu.sync_copy(obuf, out.at[pl.ds(base + c * B, B)])

    issue(0, u0, m0, su0, sm0)

    def pair_body(j, carry):
        c = 2 * j
        issue(c + 1, u1, m1, su1, sm1)
        wait(u0, m0, su0, sm0)
        compute(c, u0, m0)
        issue(c + 2, u0, m0, su0, sm0)
        wait(u1, m1, su1, sm1)
        compute(c + 1, u1, m1)
        return carry

    lax.fori_loop(0, (chunks - 1) // 2, pair_body, 0)
    wait(u0, m0, su0, sm0)
    compute(chunks - 1, u0, m0)


def kernel(x_user, x_movie, edge_label_index):
    n_edges = edge_label_index.shape[1]
    epw = n_edges // NW
    mesh = plsc.VectorSubcoreMesh(core_axis_name="c", subcore_axis_name="s")
    f = pl.kernel(
        _ip_body,
        out_type=jax.ShapeDtypeStruct((n_edges,), jnp.float32),
        mesh=mesh,
        compiler_params=pltpu.CompilerParams(needs_layout_passes=False),
        scratch_types=[
            pltpu.VMEM((epw,), jnp.int32),
            pltpu.VMEM((epw,), jnp.int32),
            pltpu.VMEM((B, D), jnp.float32),
            pltpu.VMEM((B, D), jnp.float32),
            pltpu.VMEM((B, D), jnp.float32),
            pltpu.VMEM((B, D), jnp.float32),
            pltpu.VMEM((B,), jnp.float32),
            pltpu.SemaphoreType.DMA,
            pltpu.SemaphoreType.DMA,
            pltpu.SemaphoreType.DMA,
            pltpu.SemaphoreType.DMA,
        ],
    )
    return f(x_user, x_movie, edge_label_index.reshape(2 * n_edges))


# speedup vs baseline: 1.5900x; 1.4484x over previous
import jax
import jax.numpy as jnp
from jax import lax
from jax.experimental import pallas as pl
from jax.experimental.pallas import tpu as pltpu
from jax.experimental.pallas import tpu_sc as plsc

_GDN = lax.GatherDimensionNumbers(
    offset_dims=(), collapsed_slice_dims=(0,), start_index_map=(0,))

D = 128
L = 16
NC = 2
NS = 16
NW = NC * NS
B = 80


def _ip_body(xu, xm, eidx, out, idxu_all, idxm_all,
             u0, m0, u1, m1, obuf, tmat,
             su0, sm0, su1, sm1):
    wid = lax.axis_index("s") * NC + lax.axis_index("c")
    n_edges = out.shape[0]
    epw = n_edges // NW
    chunks = epw // B
    base = wid * epw

    pltpu.sync_copy(eidx.at[pl.ds(base, epw)], idxu_all)
    pltpu.sync_copy(eidx.at[pl.ds(n_edges + base, epw)], idxm_all)

    def issue(c, ub, mb, su, sm):
        o = c * B
        pltpu.async_copy(xu.at[idxu_all.at[pl.ds(o, B)]], ub, su)
        pltpu.async_copy(xm.at[idxm_all.at[pl.ds(o, B)]], mb, sm)

    def wait(ub, mb, su, sm):
        pltpu.make_async_copy(xu.at[idxu_all.at[pl.ds(0, B)]], ub, su).wait()
        pltpu.make_async_copy(xm.at[idxm_all.at[pl.ds(0, B)]], mb, sm).wait()

    lane = lax.iota(jnp.int32, L)

    def compute(c, ub, mb):
        def group_body(g, c2):
            e0 = g * L
            for t in range(L):
                e = e0 + t
                a0 = ub[e, pl.ds(0, L)] * mb[e, pl.ds(0, L)]
                a1 = ub[e, pl.ds(L, L)] * mb[e, pl.ds(L, L)]
                for k in range(2, D // L, 2):
                    a0 = a0 + ub[e, pl.ds(k * L, L)] * mb[e, pl.ds(k * L, L)]
                    a1 = a1 + ub[e, pl.ds((k + 1) * L, L)] * mb[e, pl.ds((k + 1) * L, L)]
                plsc.store_scatter(tmat, [lane, jnp.full((L,), t, jnp.int32)],
                                   a0 + a1)
            rows = [tmat[r, pl.ds(0, L)] for r in range(L)]
            while len(rows) > 1:
                rows = [rows[i] + rows[i + 1] for i in range(0, len(rows), 2)]
            obuf[pl.ds(e0, L)] = rows[0]
            return c2

        lax.fori_loop(0, B // L, group_body, 0)
        pltpu.sync_copy(obuf, out.at[pl.ds(base + c * B, B)])

    issue(0, u0, m0, su0, sm0)

    def pair_body(j, carry):
        c = 2 * j
        issue(c + 1, u1, m1, su1, sm1)
        wait(u0, m0, su0, sm0)
        compute(c, u0, m0)
        issue(c + 2, u0, m0, su0, sm0)
        wait(u1, m1, su1, sm1)
        compute(c + 1, u1, m1)
        return carry

    lax.fori_loop(0, (chunks - 1) // 2, pair_body, 0)
    wait(u0, m0, su0, sm0)
    compute(chunks - 1, u0, m0)


def kernel(x_user, x_movie, edge_label_index):
    n_edges = edge_label_index.shape[1]
    epw = n_edges // NW
    mesh = plsc.VectorSubcoreMesh(core_axis_name="c", subcore_axis_name="s")
    f = pl.kernel(
        _ip_body,
        out_type=jax.ShapeDtypeStruct((n_edges,), jnp.float32),
        mesh=mesh,
        compiler_params=pltpu.CompilerParams(needs_layout_passes=False),
        scratch_types=[
            pltpu.VMEM((epw,), jnp.int32),
            pltpu.VMEM((epw,), jnp.int32),
            pltpu.VMEM((B, D), jnp.float32),
            pltpu.VMEM((B, D), jnp.float32),
            pltpu.VMEM((B, D), jnp.float32),
            pltpu.VMEM((B, D), jnp.float32),
            pltpu.VMEM((B,), jnp.float32),
            pltpu.VMEM((L, 17), jnp.float32),
            pltpu.SemaphoreType.DMA,
            pltpu.SemaphoreType.DMA,
            pltpu.SemaphoreType.DMA,
            pltpu.SemaphoreType.DMA,
        ],
    )
    return f(x_user, x_movie, edge_label_index.reshape(2 * n_edges))
